# baseline (speedup 1.0000x reference)
import functools

import jax
import jax.numpy as jnp
from jax import lax
from jax.experimental import pallas as pl
from jax.experimental.pallas import tpu as pltpu
from jax.experimental.pallas import tpu_sc as plsc

_PERIOD = 4096
_ROWS = 16384
_D = 2048
_NC = 2
_NS = 16
_NW = _NC * _NS
_ROWS_PER_W = _ROWS // _NW
_W_PER_GROUP = _PERIOD // _ROWS_PER_W
_B = 8
_NBUF = 6
_RAHEAD = 3
_STEPS = _ROWS_PER_W // _B


@functools.partial(
    pl.kernel,
    mesh=plsc.VectorSubcoreMesh(core_axis_name="c", subcore_axis_name="s"),
    out_type=jax.ShapeDtypeStruct((_ROWS // _PERIOD, _PERIOD, _D), jnp.float32),
    scratch_types=(
        [pltpu.VMEM((_B, _D), jnp.float32) for _ in range(_NBUF)]
        + [pltpu.SemaphoreType.DMA for _ in range(2 * _NBUF)]
    ),
)
def _gather_view(x_hbm, out_hbm, *scratch):
    bufs = scratch[:_NBUF]
    rsems = scratch[_NBUF:2 * _NBUF]
    wsems = scratch[2 * _NBUF:]
    wid = lax.axis_index("s") * _NC + lax.axis_index("c")
    g = wid // _W_PER_GROUP
    off = (wid % _W_PER_GROUP) * _ROWS_PER_W
    base = wid * _ROWS_PER_W

    def read(i):
        return pltpu.make_async_copy(
            x_hbm.at[pl.ds(base + i * _B, _B)], bufs[i % _NBUF], rsems[i % _NBUF])

    def write(i):
        return pltpu.make_async_copy(
            bufs[i % _NBUF], out_hbm.at[g, pl.ds(off + i * _B, _B)], wsems[i % _NBUF])

    waited = set()
    for i in range(_RAHEAD):
        read(i).start()
    for i in range(_STEPS):
        read(i).wait()
        write(i).start()
        nxt = i + _RAHEAD
        if nxt < _STEPS:
            j = nxt - _NBUF
            if j >= 0:
                write(j).wait()
                waited.add(j)
            read(nxt).start()
    for i in range(_STEPS):
        if i not in waited:
            write(i).wait()


def kernel(x):
    return _gather_view(x)


# speedup vs baseline: 1.0425x; 1.0425x over previous
import functools

import jax
import jax.numpy as jnp
from jax import lax
from jax.experimental import pallas as pl
from jax.experimental.pallas import tpu as pltpu
from jax.experimental.pallas import tpu_sc as plsc

_PERIOD = 4096
_ROWS = 16384
_D = 2048
_NC = 2
_NS = 16
_NW = _NC * _NS
_ROWS_PER_W = _ROWS // _NW
_W_PER_GROUP = _PERIOD // _ROWS_PER_W
_B = 8
_NBUF = 6
_RAHEAD = 3
_STEPS = _ROWS_PER_W // _B


@functools.partial(
    pl.kernel,
    mesh=plsc.VectorSubcoreMesh(core_axis_name="c", subcore_axis_name="s"),
    out_type=jax.ShapeDtypeStruct((_ROWS // _PERIOD, _PERIOD, _D), jnp.float32),
    scratch_types=(
        [pltpu.VMEM_SHARED((_NS, _NBUF, _B, _D), jnp.float32)]
        + [pltpu.SemaphoreType.DMA for _ in range(2 * _NBUF)]
    ),
)
def _gather_view_spmem(x_hbm, out_hbm, shared, *sems):
    rsems = sems[:_NBUF]
    wsems = sems[_NBUF:]
    c = lax.axis_index("c")
    s = lax.axis_index("s")
    wid = s * _NC + c
    g = wid // _W_PER_GROUP
    off = (wid % _W_PER_GROUP) * _ROWS_PER_W
    base = wid * _ROWS_PER_W

    def read(i):
        return pltpu.make_async_copy(
            x_hbm.at[pl.ds(base + i * _B, _B)],
            shared.at[s, i % _NBUF], rsems[i % _NBUF])

    def write(i):
        return pltpu.make_async_copy(
            shared.at[s, i % _NBUF],
            out_hbm.at[g, pl.ds(off + i * _B, _B)], wsems[i % _NBUF])

    waited = set()
    for i in range(_RAHEAD):
        read(i).start()
    for i in range(_STEPS):
        read(i).wait()
        write(i).start()
        nxt = i + _RAHEAD
        if nxt < _STEPS:
            j = nxt - _NBUF
            if j >= 0:
                write(j).wait()
                waited.add(j)
            read(nxt).start()
    for i in range(_STEPS):
        if i not in waited:
            write(i).wait()


def kernel(x):
    return _gather_view_spmem(x)
